# baseline (speedup 1.0000x reference)
import functools

import jax
import jax.numpy as jnp
from jax import lax
from jax.experimental import pallas as pl
from jax.experimental.pallas import tpu as pltpu
from jax.experimental.pallas import tpu_sc as plsc

B = 16384
D = 32
GROUP = 128 // D
W = 128


def _sc_gather(ut128, it128, u_group, i_group):
    mesh = plsc.VectorSubcoreMesh(core_axis_name="c", subcore_axis_name="s")

    @functools.partial(
        pl.kernel,
        mesh=mesh,
        out_type=[
            jax.ShapeDtypeStruct((B, 128), jnp.float32),
            jax.ShapeDtypeStruct((B, 128), jnp.float32),
        ],
    )
    def gather_kernel(ut_hbm, it_hbm, ui_hbm, ii_hbm, ou_hbm, oi_hbm):
        def u_body(i_vmem, o_vmem):
            pltpu.sync_copy(ut_hbm.at[i_vmem.at[0]], o_vmem)

        def i_body(i_vmem, o_vmem):
            pltpu.sync_copy(it_hbm.at[i_vmem.at[0]], o_vmem)

        for body, idx_hbm, out_hbm in ((u_body, ui_hbm, ou_hbm),
                                       (i_body, ii_hbm, oi_hbm)):
            pltpu.emit_pipeline(
                body,
                grid=(B // W,),
                in_specs=[pl.BlockSpec((1, W), lambda i: (0, i))],
                out_specs=[pl.BlockSpec((W, 128), lambda i: (i, 0))],
                core_axis_name=("c", "s"),
                dimension_semantics=(pltpu.PARALLEL,),
            )(idx_hbm, out_hbm)

    return gather_kernel(ut128, it128, u_group, i_group)


RB = 2048


def _select(x128, sub):
    out = jnp.where(sub == 0, x128[:, 0:D], 0.0)
    for k in range(1, GROUP):
        out = out + jnp.where(sub == k, x128[:, k * D:(k + 1) * D], 0.0)
    return out


def _mlp_body(u_ref, i_ref, su_ref, si_ref, w1u_ref, w1i_ref, b1_ref,
              w2_ref, b2_ref, w3_ref, b3_ref, o_ref):
    hp = lax.Precision.HIGHEST
    u = _select(u_ref[...], su_ref[...])
    i = _select(i_ref[...], si_ref[...])
    h = jnp.dot(u, w1u_ref[...], precision=hp,
                preferred_element_type=jnp.float32)
    h = h + jnp.dot(i, w1i_ref[...], precision=hp,
                    preferred_element_type=jnp.float32)
    h = jnp.maximum(h + b1_ref[...], 0.0)
    h2 = jnp.dot(h, w2_ref[...], precision=hp,
                 preferred_element_type=jnp.float32)
    h2 = jnp.maximum(h2 + b2_ref[...], 0.0)
    o = jnp.dot(h2, w3_ref[...], precision=hp,
                preferred_element_type=jnp.float32)
    o_ref[...] = o + b3_ref[...]


def _tc_mlp(u128, i128, su, si, W1, b1, W2, b2, W3, b3):
    w1u, w1i = W1[:D], W1[D:]
    b1r = b1.reshape(1, 64)
    b2r = b2.reshape(1, 32)
    b3r = b3.reshape(1, 1)
    const = lambda shape: pl.BlockSpec(shape, lambda i: (0, 0))
    return pl.pallas_call(
        _mlp_body,
        grid=(B // RB,),
        in_specs=[
            pl.BlockSpec((RB, 128), lambda i: (i, 0)),
            pl.BlockSpec((RB, 128), lambda i: (i, 0)),
            pl.BlockSpec((RB, 1), lambda i: (i, 0)),
            pl.BlockSpec((RB, 1), lambda i: (i, 0)),
            const((D, 64)),
            const((D, 64)),
            const((1, 64)),
            const((64, 32)),
            const((1, 32)),
            const((32, 1)),
            const((1, 1)),
        ],
        out_specs=pl.BlockSpec((RB, 1), lambda i: (i, 0)),
        out_shape=jax.ShapeDtypeStruct((B, 1), jnp.float32),
    )(u128, i128, su, si, w1u, w1i, b1r, W2, b2r, W3, b3r)


def kernel(user_indices, item_indices, user_table, item_table,
           W1, b1, W2, b2, W3, b3):
    ui = user_indices.astype(jnp.int32)
    ii = item_indices.astype(jnp.int32)
    nu, ni = user_table.shape[0], item_table.shape[0]
    ut128 = user_table.reshape(nu * D // 128, 128)
    it128 = item_table.reshape(ni * D // 128, 128)
    u128, i128 = _sc_gather(ut128, it128,
                            (ui // GROUP).reshape(1, B),
                            (ii // GROUP).reshape(1, B))
    su = (ui % GROUP).reshape(B, 1)
    si = (ii % GROUP).reshape(B, 1)
    return _tc_mlp(u128, i128, su, si, W1, b1, W2, b2, W3, b3)


# speedup vs baseline: 1.5760x; 1.5760x over previous
import functools

import jax
import jax.numpy as jnp
from jax import lax
from jax.experimental import pallas as pl
from jax.experimental.pallas import tpu as pltpu
from jax.experimental.pallas import tpu_sc as plsc

B = 16384
D = 32
GROUP = 128 // D
W = 128


def _sc_gather(ut128, it128, u_group, i_group):
    mesh = plsc.VectorSubcoreMesh(core_axis_name="c", subcore_axis_name="s")

    @functools.partial(
        pl.kernel,
        mesh=mesh,
        out_type=[
            jax.ShapeDtypeStruct((B, 128), jnp.float32),
            jax.ShapeDtypeStruct((B, 128), jnp.float32),
        ],
    )
    def gather_kernel(ut_hbm, it_hbm, ui_hbm, ii_hbm, ou_hbm, oi_hbm):
        def u_body(i_vmem, o_vmem):
            pltpu.sync_copy(ut_hbm.at[i_vmem.at[0]], o_vmem)

        def i_body(i_vmem, o_vmem):
            pltpu.sync_copy(it_hbm.at[i_vmem.at[0]], o_vmem)

        for body, idx_hbm, out_hbm in ((u_body, ui_hbm, ou_hbm),
                                       (i_body, ii_hbm, oi_hbm)):
            pltpu.emit_pipeline(
                body,
                grid=(B // W,),
                in_specs=[pl.BlockSpec((1, W), lambda i: (0, i))],
                out_specs=[pl.BlockSpec((W, 128), lambda i: (i, 0))],
                core_axis_name=("c", "s"),
                dimension_semantics=(pltpu.PARALLEL,),
            )(idx_hbm, out_hbm)

    return gather_kernel(ut128, it128, u_group, i_group)


RC = 2048


def _chunk(n):
    nblk = -(-(n // GROUP) // RC)
    return nblk * RC


def _repack_body(t0_ref, t1_ref, t2_ref, t3_ref, o_ref):
    refs = (t0_ref, t1_ref, t2_ref, t3_ref)
    for k in range(GROUP):
        o_ref[:, k * D:(k + 1) * D] = refs[k][...].T


def _tc_repack(tT):
    n = tT.shape[1]
    c = _chunk(n)
    nblk = c // RC
    last = -(-n // RC) - 1
    specs = [
        pl.BlockSpec((D, RC),
                     lambda i, k=k: (0, jnp.minimum(i + k * nblk, last)))
        for k in range(GROUP)
    ]
    return pl.pallas_call(
        _repack_body,
        grid=(nblk,),
        in_specs=specs,
        out_specs=pl.BlockSpec((RC, 128), lambda i: (i, 0)),
        out_shape=jax.ShapeDtypeStruct((c, 128), jnp.float32),
    )(tT, tT, tT, tT)


RB = 2048


def _select(x128, sub):
    out = jnp.where(sub == 0, x128[:, 0:D], 0.0)
    for k in range(1, GROUP):
        out = out + jnp.where(sub == k, x128[:, k * D:(k + 1) * D], 0.0)
    return out


def _mlp_body(u_ref, i_ref, su_ref, si_ref, w1u_ref, w1i_ref, b1_ref,
              w2_ref, b2_ref, w3_ref, b3_ref, o_ref):
    hp = lax.Precision.HIGHEST
    u = _select(u_ref[...], su_ref[...])
    i = _select(i_ref[...], si_ref[...])
    h = jnp.dot(u, w1u_ref[...], precision=hp,
                preferred_element_type=jnp.float32)
    h = h + jnp.dot(i, w1i_ref[...], precision=hp,
                    preferred_element_type=jnp.float32)
    h = jnp.maximum(h + b1_ref[...], 0.0)
    h2 = jnp.dot(h, w2_ref[...], precision=hp,
                 preferred_element_type=jnp.float32)
    h2 = jnp.maximum(h2 + b2_ref[...], 0.0)
    o = jnp.dot(h2, w3_ref[...], precision=hp,
                preferred_element_type=jnp.float32)
    o_ref[...] = o + b3_ref[...]


def _tc_mlp(u128, i128, su, si, W1, b1, W2, b2, W3, b3):
    w1u, w1i = W1[:D], W1[D:]
    b1r = b1.reshape(1, 64)
    b2r = b2.reshape(1, 32)
    b3r = b3.reshape(1, 1)
    const = lambda shape: pl.BlockSpec(shape, lambda i: (0, 0))
    return pl.pallas_call(
        _mlp_body,
        grid=(B // RB,),
        in_specs=[
            pl.BlockSpec((RB, 128), lambda i: (i, 0)),
            pl.BlockSpec((RB, 128), lambda i: (i, 0)),
            pl.BlockSpec((RB, 1), lambda i: (i, 0)),
            pl.BlockSpec((RB, 1), lambda i: (i, 0)),
            const((D, 64)),
            const((D, 64)),
            const((1, 64)),
            const((64, 32)),
            const((1, 32)),
            const((32, 1)),
            const((1, 1)),
        ],
        out_specs=pl.BlockSpec((RB, 1), lambda i: (i, 0)),
        out_shape=jax.ShapeDtypeStruct((B, 1), jnp.float32),
    )(u128, i128, su, si, w1u, w1i, b1r, W2, b2r, W3, b3r)


def kernel(user_indices, item_indices, user_table, item_table,
           W1, b1, W2, b2, W3, b3):
    ui = user_indices.astype(jnp.int32)
    ii = item_indices.astype(jnp.int32)
    ut128 = _tc_repack(user_table.T)
    it128 = _tc_repack(item_table.T)
    cu = _chunk(user_table.shape[0])
    ci = _chunk(item_table.shape[0])
    su = jnp.minimum(ui // cu, GROUP - 1)
    si = jnp.minimum(ii // ci, GROUP - 1)
    u128, i128 = _sc_gather(ut128, it128,
                            (ui - su * cu).reshape(1, B),
                            (ii - si * ci).reshape(1, B))
    su = su.reshape(B, 1)
    si = si.reshape(B, 1)
    return _tc_mlp(u128, i128, su, si, W1, b1, W2, b2, W3, b3)


# speedup vs baseline: 3.1130x; 1.9753x over previous
import functools

import jax
import jax.numpy as jnp
from jax import lax
from jax.experimental import pallas as pl
from jax.experimental.pallas import tpu as pltpu
from jax.experimental.pallas import tpu_sc as plsc

B = 16384
D = 32
GROUP = 128 // D
W = 128


def _sc_gather(ut128, it128, u_group, i_group):
    mesh = plsc.VectorSubcoreMesh(core_axis_name="c", subcore_axis_name="s")

    @functools.partial(
        pl.kernel,
        mesh=mesh,
        out_type=[
            jax.ShapeDtypeStruct((B, 128), jnp.float32),
            jax.ShapeDtypeStruct((B, 128), jnp.float32),
        ],
    )
    def gather_kernel(ut_hbm, it_hbm, ui_hbm, ii_hbm, ou_hbm, oi_hbm):
        def u_body(i_vmem, o_vmem):
            pltpu.sync_copy(ut_hbm.at[i_vmem.at[0]], o_vmem)

        def i_body(i_vmem, o_vmem):
            pltpu.sync_copy(it_hbm.at[i_vmem.at[0]], o_vmem)

        for body, idx_hbm, out_hbm in ((u_body, ui_hbm, ou_hbm),
                                       (i_body, ii_hbm, oi_hbm)):
            pltpu.emit_pipeline(
                body,
                grid=(B // W,),
                in_specs=[pl.BlockSpec((1, W), lambda i: (0, i))],
                out_specs=[pl.BlockSpec((W, 128), lambda i: (i, 0))],
                core_axis_name=("c", "s"),
                dimension_semantics=(pltpu.PARALLEL,),
            )(idx_hbm, out_hbm)

    return gather_kernel(ut128, it128, u_group, i_group)


RC = 4096


def _chunk(n):
    nblk = -(-(n // GROUP) // RC)
    return nblk * RC


def _repack_body(t0_ref, t1_ref, t2_ref, t3_ref, o_ref):
    x = jnp.concatenate(
        [t0_ref[...], t1_ref[...], t2_ref[...], t3_ref[...]], axis=0)
    o_ref[...] = x.T


def _tc_repack(tT):
    n = tT.shape[1]
    c = _chunk(n)
    nblk = c // RC
    last = -(-n // RC) - 1
    specs = [
        pl.BlockSpec((D, RC),
                     lambda i, k=k: (0, jnp.minimum(i + k * nblk, last)))
        for k in range(GROUP)
    ]
    return pl.pallas_call(
        _repack_body,
        grid=(nblk,),
        in_specs=specs,
        out_specs=pl.BlockSpec((RC, 128), lambda i: (i, 0)),
        out_shape=jax.ShapeDtypeStruct((c, 128), jnp.float32),
    )(tT, tT, tT, tT)


RB = 2048


def _select(x128, sub):
    out = jnp.where(sub == 0, x128[:, 0:D], 0.0)
    for k in range(1, GROUP):
        out = out + jnp.where(sub == k, x128[:, k * D:(k + 1) * D], 0.0)
    return out


def _mlp_body(u_ref, i_ref, su_ref, si_ref, w1u_ref, w1i_ref, b1_ref,
              w2_ref, b2_ref, w3_ref, b3_ref, o_ref):
    hp = lax.Precision.HIGHEST
    u = _select(u_ref[...], su_ref[...])
    i = _select(i_ref[...], si_ref[...])
    h = jnp.dot(u, w1u_ref[...], precision=hp,
                preferred_element_type=jnp.float32)
    h = h + jnp.dot(i, w1i_ref[...], precision=hp,
                    preferred_element_type=jnp.float32)
    h = jnp.maximum(h + b1_ref[...], 0.0)
    h2 = jnp.dot(h, w2_ref[...], precision=hp,
                 preferred_element_type=jnp.float32)
    h2 = jnp.maximum(h2 + b2_ref[...], 0.0)
    o = jnp.dot(h2, w3_ref[...], precision=hp,
                preferred_element_type=jnp.float32)
    o_ref[...] = o + b3_ref[...]


def _tc_mlp(u128, i128, su, si, W1, b1, W2, b2, W3, b3):
    w1u, w1i = W1[:D], W1[D:]
    b1r = b1.reshape(1, 64)
    b2r = b2.reshape(1, 32)
    b3r = b3.reshape(1, 1)
    const = lambda shape: pl.BlockSpec(shape, lambda i: (0, 0))
    return pl.pallas_call(
        _mlp_body,
        grid=(B // RB,),
        in_specs=[
            pl.BlockSpec((RB, 128), lambda i: (i, 0)),
            pl.BlockSpec((RB, 128), lambda i: (i, 0)),
            pl.BlockSpec((RB, 1), lambda i: (i, 0)),
            pl.BlockSpec((RB, 1), lambda i: (i, 0)),
            const((D, 64)),
            const((D, 64)),
            const((1, 64)),
            const((64, 32)),
            const((1, 32)),
            const((32, 1)),
            const((1, 1)),
        ],
        out_specs=pl.BlockSpec((RB, 1), lambda i: (i, 0)),
        out_shape=jax.ShapeDtypeStruct((B, 1), jnp.float32),
    )(u128, i128, su, si, w1u, w1i, b1r, W2, b2r, W3, b3r)


def kernel(user_indices, item_indices, user_table, item_table,
           W1, b1, W2, b2, W3, b3):
    ui = user_indices.astype(jnp.int32)
    ii = item_indices.astype(jnp.int32)
    ut128 = _tc_repack(user_table.T)
    it128 = _tc_repack(item_table.T)
    cu = _chunk(user_table.shape[0])
    ci = _chunk(item_table.shape[0])
    su = jnp.minimum(ui // cu, GROUP - 1)
    si = jnp.minimum(ii // ci, GROUP - 1)
    u128, i128 = _sc_gather(ut128, it128,
                            (ui - su * cu).reshape(1, B),
                            (ii - si * ci).reshape(1, B))
    su = su.reshape(B, 1)
    si = si.reshape(B, 1)
    return _tc_mlp(u128, i128, su, si, W1, b1, W2, b2, W3, b3)


# speedup vs baseline: 3.9193x; 1.2590x over previous
import functools

import jax
import jax.numpy as jnp
from jax import lax
from jax.experimental import pallas as pl
from jax.experimental.pallas import tpu as pltpu
from jax.experimental.pallas import tpu_sc as plsc

B = 16384
D = 32
HALF = D // 2
GROUP = 8
W = 128
RC = 4096


def _chunk(n):
    return -(-(n // GROUP) // RC) * RC


def _repack_body(*refs):
    o_ref = refs[-1]
    parts = []
    for k in range(GROUP):
        xb = refs[k][...].astype(jnp.bfloat16)
        u = lax.bitcast_convert_type(xb, jnp.uint16).astype(jnp.uint32)
        pk = u[:HALF] | (u[HALF:] << 16)
        parts.append(lax.bitcast_convert_type(pk, jnp.float32))
    z = jnp.concatenate(parts, axis=0)
    o_ref[...] = z.T


def _tc_repack(tT):
    n = tT.shape[1]
    c = _chunk(n)
    nblk = c // RC
    last = -(-n // RC) - 1
    specs = [
        pl.BlockSpec((D, RC),
                     lambda i, k=k: (0, jnp.minimum(i + k * nblk, last)))
        for k in range(GROUP)
    ]
    return pl.pallas_call(
        _repack_body,
        grid=(nblk,),
        in_specs=specs,
        out_specs=pl.BlockSpec((RC, 128), lambda i: (i, 0)),
        out_shape=jax.ShapeDtypeStruct((c, 128), jnp.float32),
        compiler_params=pltpu.CompilerParams(
            dimension_semantics=("parallel",)),
    )(*([tT] * GROUP))


def _sc_gather(ut128, it128, u_group, i_group):
    mesh = plsc.VectorSubcoreMesh(core_axis_name="c", subcore_axis_name="s")

    @functools.partial(
        pl.kernel,
        mesh=mesh,
        out_type=[
            jax.ShapeDtypeStruct((B, 128), jnp.float32),
            jax.ShapeDtypeStruct((B, 128), jnp.float32),
        ],
    )
    def gather_kernel(ut_hbm, it_hbm, ui_hbm, ii_hbm, ou_hbm, oi_hbm):
        def u_body(i_vmem, o_vmem):
            pltpu.sync_copy(ut_hbm.at[i_vmem.at[0]], o_vmem)

        def i_body(i_vmem, o_vmem):
            pltpu.sync_copy(it_hbm.at[i_vmem.at[0]], o_vmem)

        for body, idx_hbm, out_hbm in ((u_body, ui_hbm, ou_hbm),
                                       (i_body, ii_hbm, oi_hbm)):
            pltpu.emit_pipeline(
                body,
                grid=(B // W,),
                in_specs=[pl.BlockSpec((1, W), lambda i: (0, i))],
                out_specs=[pl.BlockSpec((W, 128), lambda i: (i, 0))],
                core_axis_name=("c", "s"),
                dimension_semantics=(pltpu.PARALLEL,),
            )(idx_hbm, out_hbm)

    return gather_kernel(ut128, it128, u_group, i_group)


RB = 2048


def _select_unpack(x128, sub):
    zero = jnp.zeros((), dtype=x128.dtype)
    sel = jnp.where(sub == 0, x128[:, 0:HALF], zero)
    for k in range(1, GROUP):
        sel = sel + jnp.where(sub == k, x128[:, k * HALF:(k + 1) * HALF],
                              zero)
    bits = lax.bitcast_convert_type(sel, jnp.uint32)
    lo = lax.bitcast_convert_type(bits.astype(jnp.uint16), jnp.bfloat16)
    hi = lax.bitcast_convert_type((bits >> 16).astype(jnp.uint16),
                                  jnp.bfloat16)
    return lo, hi


def _mlp_body(u_ref, i_ref, su_ref, si_ref, w1ul_ref, w1uh_ref, w1il_ref,
              w1ih_ref, b1_ref, w2_ref, b2_ref, w3_ref, b3_ref, o_ref):
    ulo, uhi = _select_unpack(u_ref[...], su_ref[...])
    ilo, ihi = _select_unpack(i_ref[...], si_ref[...])
    h = jnp.dot(ulo, w1ul_ref[...], preferred_element_type=jnp.float32)
    h += jnp.dot(uhi, w1uh_ref[...], preferred_element_type=jnp.float32)
    h += jnp.dot(ilo, w1il_ref[...], preferred_element_type=jnp.float32)
    h += jnp.dot(ihi, w1ih_ref[...], preferred_element_type=jnp.float32)
    h = jnp.maximum(h + b1_ref[...], 0.0).astype(jnp.bfloat16)
    h2 = jnp.dot(h, w2_ref[...], preferred_element_type=jnp.float32)
    h2 = jnp.maximum(h2 + b2_ref[...], 0.0).astype(jnp.bfloat16)
    o = jnp.dot(h2, w3_ref[...], preferred_element_type=jnp.float32)
    o_ref[...] = o + b3_ref[...]


def _tc_mlp(u128, i128, su, si, W1, b1, W2, b2, W3, b3):
    w1 = W1.astype(jnp.bfloat16)
    w1ul, w1uh = w1[0:HALF], w1[HALF:D]
    w1il, w1ih = w1[D:D + HALF], w1[D + HALF:]
    w2 = W2.astype(jnp.bfloat16)
    w3 = W3.astype(jnp.bfloat16)
    b1r = b1.reshape(1, 64)
    b2r = b2.reshape(1, 32)
    b3r = b3.reshape(1, 1)
    const = lambda shape: pl.BlockSpec(shape, lambda i: (0, 0))
    return pl.pallas_call(
        _mlp_body,
        grid=(B // RB,),
        in_specs=[
            pl.BlockSpec((RB, 128), lambda i: (i, 0)),
            pl.BlockSpec((RB, 128), lambda i: (i, 0)),
            pl.BlockSpec((RB, 1), lambda i: (i, 0)),
            pl.BlockSpec((RB, 1), lambda i: (i, 0)),
            const((HALF, 64)),
            const((HALF, 64)),
            const((HALF, 64)),
            const((HALF, 64)),
            const((1, 64)),
            const((64, 32)),
            const((1, 32)),
            const((32, 1)),
            const((1, 1)),
        ],
        out_specs=pl.BlockSpec((RB, 1), lambda i: (i, 0)),
        out_shape=jax.ShapeDtypeStruct((B, 1), jnp.float32),
        compiler_params=pltpu.CompilerParams(
            dimension_semantics=("parallel",)),
    )(u128, i128, su, si, w1ul, w1uh, w1il, w1ih, b1r, w2, b2r, w3, b3r)


def kernel(user_indices, item_indices, user_table, item_table,
           W1, b1, W2, b2, W3, b3):
    ui = user_indices.astype(jnp.int32)
    ii = item_indices.astype(jnp.int32)
    ut128 = _tc_repack(user_table.T)
    it128 = _tc_repack(item_table.T)
    cu = _chunk(user_table.shape[0])
    ci = _chunk(item_table.shape[0])
    su = jnp.minimum(ui // cu, GROUP - 1)
    si = jnp.minimum(ii // ci, GROUP - 1)
    u128, i128 = _sc_gather(ut128, it128,
                            (ui - su * cu).reshape(1, B),
                            (ii - si * ci).reshape(1, B))
    return _tc_mlp(u128, i128, su.reshape(B, 1), si.reshape(B, 1),
                   W1, b1, W2, b2, W3, b3)


# speedup vs baseline: 4.9145x; 1.2539x over previous
import functools

import jax
import jax.numpy as jnp
from jax import lax
from jax.experimental import pallas as pl
from jax.experimental.pallas import tpu as pltpu
from jax.experimental.pallas import tpu_sc as plsc

B = 16384
D = 32
HALF = D // 2
GROUP = 8
W = 128
RC = 4096


def _chunk(n):
    return -(-(n // GROUP) // RC) * RC


def _repack_body(*refs):
    o_ref = refs[-1]
    parts = []
    for k in range(GROUP):
        xb = refs[k][...].astype(jnp.bfloat16)
        u = lax.bitcast_convert_type(xb, jnp.uint16).astype(jnp.uint32)
        pk = u[:HALF] | (u[HALF:] << 16)
        parts.append(lax.bitcast_convert_type(pk, jnp.float32))
    z = jnp.concatenate(parts, axis=0)
    o_ref[...] = z.T


def _tc_repack(tT):
    n = tT.shape[1]
    c = _chunk(n)
    nblk = c // RC
    last = -(-n // RC) - 1
    specs = [
        pl.BlockSpec((D, RC),
                     lambda i, k=k: (0, jnp.minimum(i + k * nblk, last)))
        for k in range(GROUP)
    ]
    return pl.pallas_call(
        _repack_body,
        grid=(nblk,),
        in_specs=specs,
        out_specs=pl.BlockSpec((RC, 128), lambda i: (i, 0)),
        out_shape=jax.ShapeDtypeStruct((c, 128), jnp.float32),
        compiler_params=pltpu.CompilerParams(
            dimension_semantics=("parallel",)),
    )(*([tT] * GROUP))


def _sc_gather(t128, group_idx):
    mesh = plsc.VectorSubcoreMesh(core_axis_name="c", subcore_axis_name="s")

    @functools.partial(
        pl.kernel,
        mesh=mesh,
        out_type=jax.ShapeDtypeStruct((B, 128), jnp.float32),
    )
    def gather_kernel(t_hbm, idx_hbm, out_hbm):
        def body(i_vmem, o_vmem):
            pltpu.sync_copy(t_hbm.at[i_vmem.at[0]], o_vmem)

        pltpu.emit_pipeline(
            body,
            grid=(B // W,),
            in_specs=[pl.BlockSpec((1, W), lambda i: (0, i))],
            out_specs=[pl.BlockSpec((W, 128), lambda i: (i, 0))],
            core_axis_name=("c", "s"),
            dimension_semantics=(pltpu.PARALLEL,),
        )(idx_hbm, out_hbm)

    return gather_kernel(t128, group_idx)


RB = 4096


def _select_unpack(x128, sub):
    cols = sub * HALF + lax.broadcasted_iota(jnp.int32, (1, HALF), 1)
    sel = jnp.take_along_axis(x128, cols, axis=1)
    bits = lax.bitcast_convert_type(sel, jnp.uint32)
    lo = lax.bitcast_convert_type(bits.astype(jnp.uint16), jnp.bfloat16)
    hi = lax.bitcast_convert_type((bits >> 16).astype(jnp.uint16),
                                  jnp.bfloat16)
    return lo, hi


def _mlp_body(u_ref, i_ref, su_ref, si_ref, w1ul_ref, w1uh_ref, w1il_ref,
              w1ih_ref, b1_ref, w2_ref, b2_ref, w3_ref, b3_ref, o_ref):
    ulo, uhi = _select_unpack(u_ref[...], su_ref[...])
    ilo, ihi = _select_unpack(i_ref[...], si_ref[...])
    h = jnp.dot(ulo, w1ul_ref[...], preferred_element_type=jnp.float32)
    h += jnp.dot(uhi, w1uh_ref[...], preferred_element_type=jnp.float32)
    h += jnp.dot(ilo, w1il_ref[...], preferred_element_type=jnp.float32)
    h += jnp.dot(ihi, w1ih_ref[...], preferred_element_type=jnp.float32)
    h = jnp.maximum(h + b1_ref[...], 0.0).astype(jnp.bfloat16)
    h2 = jnp.dot(h, w2_ref[...], preferred_element_type=jnp.float32)
    h2 = jnp.maximum(h2 + b2_ref[...], 0.0).astype(jnp.bfloat16)
    o = jnp.dot(h2, w3_ref[...], preferred_element_type=jnp.float32)
    o_ref[...] = o + b3_ref[...]


def _tc_mlp(u128, i128, su, si, W1, b1, W2, b2, W3, b3):
    w1 = W1.astype(jnp.bfloat16)
    w1ul, w1uh = w1[0:HALF], w1[HALF:D]
    w1il, w1ih = w1[D:D + HALF], w1[D + HALF:]
    w2 = W2.astype(jnp.bfloat16)
    w3 = W3.astype(jnp.bfloat16)
    b1r = b1.reshape(1, 64)
    b2r = b2.reshape(1, 32)
    b3r = b3.reshape(1, 1)
    const = lambda shape: pl.BlockSpec(shape, lambda i: (0, 0))
    return pl.pallas_call(
        _mlp_body,
        grid=(B // RB,),
        in_specs=[
            pl.BlockSpec((RB, 128), lambda i: (i, 0)),
            pl.BlockSpec((RB, 128), lambda i: (i, 0)),
            pl.BlockSpec((RB, 1), lambda i: (i, 0)),
            pl.BlockSpec((RB, 1), lambda i: (i, 0)),
            const((HALF, 64)),
            const((HALF, 64)),
            const((HALF, 64)),
            const((HALF, 64)),
            const((1, 64)),
            const((64, 32)),
            const((1, 32)),
            const((32, 1)),
            const((1, 1)),
        ],
        out_specs=pl.BlockSpec((RB, 1), lambda i: (i, 0)),
        out_shape=jax.ShapeDtypeStruct((B, 1), jnp.float32),
        compiler_params=pltpu.CompilerParams(
            dimension_semantics=("parallel",)),
    )(u128, i128, su, si, w1ul, w1uh, w1il, w1ih, b1r, w2, b2r, w3, b3r)


def kernel(user_indices, item_indices, user_table, item_table,
           W1, b1, W2, b2, W3, b3):
    ui = user_indices.astype(jnp.int32)
    ii = item_indices.astype(jnp.int32)
    cu = _chunk(user_table.shape[0])
    ci = _chunk(item_table.shape[0])
    su = jnp.minimum(ui // cu, GROUP - 1)
    si = jnp.minimum(ii // ci, GROUP - 1)
    ut128 = _tc_repack(user_table.T)
    u128 = _sc_gather(ut128, (ui - su * cu).reshape(1, B))
    it128 = _tc_repack(item_table.T)
    i128 = _sc_gather(it128, (ii - si * ci).reshape(1, B))
    return _tc_mlp(u128, i128, su.reshape(B, 1), si.reshape(B, 1),
                   W1, b1, W2, b2, W3, b3)


# speedup vs baseline: 5.2349x; 1.0652x over previous
import functools

import jax
import jax.numpy as jnp
from jax import lax
from jax.experimental import pallas as pl
from jax.experimental.pallas import tpu as pltpu
from jax.experimental.pallas import tpu_sc as plsc

B = 16384
D = 32
HALF = D // 2
GROUP = 8
W = 128
RC = 8192


def _chunk(n):
    return -(-(n // GROUP) // RC) * RC


def _repack_body(*refs):
    o_ref = refs[-1]
    parts = []
    for k in range(GROUP):
        xb = refs[k][...].astype(jnp.bfloat16)
        u = lax.bitcast_convert_type(xb, jnp.uint16).astype(jnp.uint32)
        pk = u[:HALF] | (u[HALF:] << 16)
        parts.append(lax.bitcast_convert_type(pk, jnp.float32))
    z = jnp.concatenate(parts, axis=0)
    o_ref[...] = z.T


def _tc_repack(tT):
    n = tT.shape[1]
    c = _chunk(n)
    nblk = c // RC
    last = -(-n // RC) - 1
    specs = [
        pl.BlockSpec((D, RC),
                     lambda i, k=k: (0, jnp.minimum(i + k * nblk, last)))
        for k in range(GROUP)
    ]
    return pl.pallas_call(
        _repack_body,
        grid=(nblk,),
        in_specs=specs,
        out_specs=pl.BlockSpec((RC, 128), lambda i: (i, 0)),
        out_shape=jax.ShapeDtypeStruct((c, 128), jnp.float32),
        compiler_params=pltpu.CompilerParams(
            dimension_semantics=("parallel",)),
    )(*([tT] * GROUP))


def _sc_gather(t128, group_idx):
    mesh = plsc.VectorSubcoreMesh(core_axis_name="c", subcore_axis_name="s")

    @functools.partial(
        pl.kernel,
        mesh=mesh,
        out_type=jax.ShapeDtypeStruct((B, 128), jnp.float32),
    )
    def gather_kernel(t_hbm, idx_hbm, out_hbm):
        def body(i_vmem, o_vmem):
            pltpu.sync_copy(t_hbm.at[i_vmem.at[0]], o_vmem)

        pltpu.emit_pipeline(
            body,
            grid=(B // W,),
            in_specs=[pl.BlockSpec((1, W), lambda i: (0, i))],
            out_specs=[pl.BlockSpec((W, 128), lambda i: (i, 0))],
            core_axis_name=("c", "s"),
            dimension_semantics=(pltpu.PARALLEL,),
        )(idx_hbm, out_hbm)

    return gather_kernel(t128, group_idx)


RB = 4096


def _select_unpack(x128, sub):
    cols = sub * HALF + lax.broadcasted_iota(jnp.int32, (1, HALF), 1)
    sel = jnp.take_along_axis(x128, cols, axis=1)
    bits = lax.bitcast_convert_type(sel, jnp.uint32)
    lo = lax.bitcast_convert_type(bits.astype(jnp.uint16), jnp.bfloat16)
    hi = lax.bitcast_convert_type((bits >> 16).astype(jnp.uint16),
                                  jnp.bfloat16)
    return lo, hi


def _mlp_body(u_ref, i_ref, su_ref, si_ref, w1ul_ref, w1uh_ref, w1il_ref,
              w1ih_ref, b1_ref, w2_ref, b2_ref, w3_ref, b3_ref, o_ref):
    ulo, uhi = _select_unpack(u_ref[...], su_ref[...])
    ilo, ihi = _select_unpack(i_ref[...], si_ref[...])
    h = jnp.dot(ulo, w1ul_ref[...], preferred_element_type=jnp.float32)
    h += jnp.dot(uhi, w1uh_ref[...], preferred_element_type=jnp.float32)
    h += jnp.dot(ilo, w1il_ref[...], preferred_element_type=jnp.float32)
    h += jnp.dot(ihi, w1ih_ref[...], preferred_element_type=jnp.float32)
    h = jnp.maximum(h + b1_ref[...], 0.0).astype(jnp.bfloat16)
    h2 = jnp.dot(h, w2_ref[...], preferred_element_type=jnp.float32)
    h2 = jnp.maximum(h2 + b2_ref[...], 0.0).astype(jnp.bfloat16)
    o = jnp.dot(h2, w3_ref[...], preferred_element_type=jnp.float32)
    o_ref[...] = (o + b3_ref[...]).T


def _tc_mlp(u128, i128, su, si, W1, b1, W2, b2, W3, b3):
    w1 = W1.astype(jnp.bfloat16)
    w1ul, w1uh = w1[0:HALF], w1[HALF:D]
    w1il, w1ih = w1[D:D + HALF], w1[D + HALF:]
    w2 = W2.astype(jnp.bfloat16)
    w3 = W3.astype(jnp.bfloat16)
    b1r = b1.reshape(1, 64)
    b2r = b2.reshape(1, 32)
    b3r = b3.reshape(1, 1)
    const = lambda shape: pl.BlockSpec(shape, lambda i: (0, 0))
    return pl.pallas_call(
        _mlp_body,
        grid=(B // RB,),
        in_specs=[
            pl.BlockSpec((RB, 128), lambda i: (i, 0)),
            pl.BlockSpec((RB, 128), lambda i: (i, 0)),
            pl.BlockSpec((RB, 1), lambda i: (i, 0)),
            pl.BlockSpec((RB, 1), lambda i: (i, 0)),
            const((HALF, 64)),
            const((HALF, 64)),
            const((HALF, 64)),
            const((HALF, 64)),
            const((1, 64)),
            const((64, 32)),
            const((1, 32)),
            const((32, 1)),
            const((1, 1)),
        ],
        out_specs=pl.BlockSpec((1, RB), lambda i: (0, i)),
        out_shape=jax.ShapeDtypeStruct((1, B), jnp.float32),
        compiler_params=pltpu.CompilerParams(
            dimension_semantics=("parallel",)),
    )(u128, i128, su, si, w1ul, w1uh, w1il, w1ih, b1r, w2, b2r, w3, b3r)


def kernel(user_indices, item_indices, user_table, item_table,
           W1, b1, W2, b2, W3, b3):
    ui = user_indices.astype(jnp.int32)
    ii = item_indices.astype(jnp.int32)
    cu = _chunk(user_table.shape[0])
    ci = _chunk(item_table.shape[0])
    su = jnp.minimum(ui // cu, GROUP - 1)
    si = jnp.minimum(ii // ci, GROUP - 1)
    ut128 = _tc_repack(user_table.T)
    u128 = _sc_gather(ut128, (ui - su * cu).reshape(1, B))
    it128 = _tc_repack(item_table.T)
    i128 = _sc_gather(it128, (ii - si * ci).reshape(1, B))
    out = _tc_mlp(u128, i128, su.reshape(B, 1), si.reshape(B, 1),
                  W1, b1, W2, b2, W3, b3)
    return out.T
